# baseline (speedup 1.0000x reference)
import functools
import math

import jax
import jax.numpy as jnp
from jax import lax
from jax.experimental import pallas as pl
from jax.experimental.pallas import tpu as pltpu
from jax.experimental.pallas import tpu_sc as plsc

_D = 1048576
_N = 10
_DBLK = 8192
_NBLK = _D // _DBLK
_LN4 = math.log(4.0)

_NW = 32
_RPW = _D // _NW
_CH = 8192


def _stage1_body(x_ref, w_ref, gacc, sacc):
    i = pl.program_id(0)
    blk = x_ref[...]
    bb = blk.astype(jnp.bfloat16)
    g = lax.dot_general(bb, bb, (((0,), (0,)), ((), ())),
                        preferred_element_type=jnp.float32)
    sq = jnp.sum(blk * blk, axis=0, keepdims=True)

    @pl.when(i == 0)
    def _():
        gacc[...] = g
        sacc[...] = sq

    @pl.when(i > 0)
    def _():
        gacc[...] = gacc[...] + g
        sacc[...] = sacc[...] + sq

    @pl.when(i == _NBLK - 1)
    def _():
        G = gacc[...]
        sn = sacc[...]
        lane10 = lax.broadcasted_iota(jnp.int32, (1, _N), 1)
        lanes2 = lax.broadcasted_iota(jnp.int32, (_N, _N), 1)
        rows2 = lax.broadcasted_iota(jnp.int32, (_N, _N), 0)
        rowc = lax.broadcasted_iota(jnp.int32, (_N, 1), 0)
        BIGF = jnp.float32(3.0e38)
        BIGI = jnp.int32(999)

        sncol = jnp.zeros((_N, _N), jnp.float32)
        for k in range(_N):
            snk = jnp.sum(jnp.where(lane10 == k, sn, 0.0))
            sncol = sncol + jnp.where(rows2 == k, snk, 0.0)
        snrow = jnp.broadcast_to(sn, (_N, _N))
        sqm = (sncol + snrow) - 2.0 * G
        cd = jnp.sqrt(jnp.maximum(sqm, jnp.float32(1e-12)))

        cur_mask = lanes2 >= 0
        vals = []
        idxs = []
        for k in range(4):
            curv = jnp.where(cur_mask, cd, BIGF)
            mk = jnp.min(curv, axis=1, keepdims=True)
            jk = jnp.min(jnp.where(curv == mk, lanes2, BIGI),
                         axis=1, keepdims=True)
            cur_mask = cur_mask & (lanes2 != jk)
            vals.append(mk)
            idxs.append(jk)
        s = ((vals[0] + vals[1]) + vals[2]) + vals[3]

        smin = jnp.min(s)
        istar = jnp.min(jnp.where(s == smin, rowc, BIGI))

        ii = []
        sc = []
        for k in range(4):
            iik = jnp.sum(jnp.where(rowc == istar, idxs[k], 0))
            ii.append(iik)
            sc.append(jnp.sum(jnp.where(rowc == iik, s, 0.0)))

        lane128 = lax.broadcasted_iota(jnp.int32, (1, 128), 1)
        m4 = lane128 < 4
        sv = jnp.zeros((1, 128), jnp.float32)
        for k in range(4):
            sv = sv + jnp.where(lane128 == k, sc[k], 0.0)

        mn = jnp.min(jnp.where(m4, sv, BIGF))
        mx = jnp.max(jnp.where(m4, sv, -BIGF))
        neq = mn != mx
        denom = jnp.where(neq, mx - mn, jnp.float32(1.0))
        nor = jnp.where(neq, (mx - sv) / denom, sv)
        nor = jnp.where(m4, nor, 0.0)
        tot = jnp.sum(nor)
        p = nor / tot
        pn0 = p != 0.0
        ent = jnp.sum(jnp.where(m4 & pn0,
                                p * jnp.log(jnp.where(pn0, p, 1.0)), 0.0))
        Es = (-1.0 / _LN4) * ent

        rep = jnp.where(m4, jnp.float32(0.05), 0.0)
        totr = jnp.sum(rep)
        q = rep / totr
        qn0 = q != 0.0
        entr = jnp.sum(jnp.where(m4 & qn0,
                                 q * jnp.log(jnp.where(qn0, q, 1.0)), 0.0))
        Er = (-1.0 / _LN4) * entr

        alpha = (1.0 - Es) / (2.0 - Es - Er)
        beta = (1.0 - Er) / (2.0 - Es - Er)
        xin = jnp.where(m4, alpha * sv + beta * rep, 0.0)
        xi = xin / jnp.sum(xin)

        w = jnp.zeros((1, 128), jnp.float32)
        for k in range(4):
            xik = jnp.sum(jnp.where(lane128 == k, xi, 0.0))
            w = w + jnp.where(lane128 == ii[k], xik, 0.0)
        w_ref[...] = w


def _stage1(x2d):
    return pl.pallas_call(
        _stage1_body,
        grid=(_NBLK,),
        in_specs=[pl.BlockSpec((_DBLK, _N), lambda i: (i, 0))],
        out_specs=pl.BlockSpec((1, 128), lambda i: (0, 0)),
        out_shape=jax.ShapeDtypeStruct((1, 128), jnp.float32),
        scratch_shapes=[pltpu.VMEM((_N, _N), jnp.float32),
                        pltpu.VMEM((1, _N), jnp.float32)],
    )(x2d)


def _stage2(x1d, wflat):
    mesh = plsc.VectorSubcoreMesh(core_axis_name="c", subcore_axis_name="s")

    @functools.partial(
        pl.kernel,
        mesh=mesh,
        out_type=jax.ShapeDtypeStruct((_D,), jnp.float32),
        compiler_params=pltpu.CompilerParams(needs_layout_passes=False),
        scratch_types=[pltpu.VMEM((_CH * _N,), jnp.float32),
                       pltpu.VMEM((_CH,), jnp.float32),
                       pltpu.VMEM((_N * 16,), jnp.float32)],
    )
    def k(x_hbm, w_hbm, out_hbm, xv, ov, wv):
        wid = lax.axis_index("s") * 2 + lax.axis_index("c")
        pltpu.sync_copy(w_hbm, wv)
        for c in range(_RPW // _CH):
            base = wid * _RPW + c * _CH
            pltpu.sync_copy(x_hbm.at[pl.ds(base * _N, _CH * _N)], xv)

            def body(gi, carry):
                flat = (gi * 16 + lax.iota(jnp.int32, 16)) * _N
                acc = None
                for j in range(_N):
                    v = plsc.load_gather(xv, [flat + j])
                    t = v * wv[pl.ds(j * 16, 16)]
                    acc = t if acc is None else acc + t
                ov[pl.ds(gi * 16, 16)] = acc
                return carry

            lax.fori_loop(0, _CH // 16, body, 0)
            pltpu.sync_copy(ov, out_hbm.at[pl.ds(base, _CH)])

    return k(x1d, wflat)


def kernel(input):
    x2d = input.reshape(_D, _N)
    w128 = _stage1(x2d)
    w10 = w128[0, :_N]
    wflat = jnp.broadcast_to(w10[:, None], (_N, 16)).reshape(_N * 16)
    out = _stage2(input.reshape(_D * _N), wflat)
    return out.reshape(1, _D, 1)


# speedup vs baseline: 1.0040x; 1.0040x over previous
import functools
import math

import jax
import jax.numpy as jnp
from jax import lax
from jax.experimental import pallas as pl
from jax.experimental.pallas import tpu as pltpu
from jax.experimental.pallas import tpu_sc as plsc

_D = 1048576
_N = 10
_DBLK = 8192
_NBLK = _D // _DBLK
_LN4 = math.log(4.0)

_NW = 32
_RPW = _D // _NW
_CH = 8192


def _stage1_body(x_ref, w_ref, gacc, sacc):
    i = pl.program_id(0)
    blk = x_ref[...]
    bb = blk.astype(jnp.bfloat16)
    g = lax.dot_general(bb, bb, (((0,), (0,)), ((), ())),
                        preferred_element_type=jnp.float32)
    sq = jnp.sum(blk * blk, axis=0, keepdims=True)

    @pl.when(i == 0)
    def _():
        gacc[...] = g
        sacc[...] = sq

    @pl.when(i > 0)
    def _():
        gacc[...] = gacc[...] + g
        sacc[...] = sacc[...] + sq

    @pl.when(i == _NBLK - 1)
    def _():
        G = gacc[...]
        sn = sacc[...]
        lane10 = lax.broadcasted_iota(jnp.int32, (1, _N), 1)
        lanes2 = lax.broadcasted_iota(jnp.int32, (_N, _N), 1)
        rows2 = lax.broadcasted_iota(jnp.int32, (_N, _N), 0)
        rowc = lax.broadcasted_iota(jnp.int32, (_N, 1), 0)
        BIGF = jnp.float32(3.0e38)
        BIGI = jnp.int32(999)

        sncol = jnp.zeros((_N, _N), jnp.float32)
        for k in range(_N):
            snk = jnp.sum(jnp.where(lane10 == k, sn, 0.0))
            sncol = sncol + jnp.where(rows2 == k, snk, 0.0)
        snrow = jnp.broadcast_to(sn, (_N, _N))
        sqm = (sncol + snrow) - 2.0 * G
        cd = jnp.sqrt(jnp.maximum(sqm, jnp.float32(1e-12)))

        cur_mask = lanes2 >= 0
        vals = []
        idxs = []
        for k in range(4):
            curv = jnp.where(cur_mask, cd, BIGF)
            mk = jnp.min(curv, axis=1, keepdims=True)
            jk = jnp.min(jnp.where(curv == mk, lanes2, BIGI),
                         axis=1, keepdims=True)
            cur_mask = cur_mask & (lanes2 != jk)
            vals.append(mk)
            idxs.append(jk)
        s = ((vals[0] + vals[1]) + vals[2]) + vals[3]

        smin = jnp.min(s)
        istar = jnp.min(jnp.where(s == smin, rowc, BIGI))

        ii = []
        sc = []
        for k in range(4):
            iik = jnp.sum(jnp.where(rowc == istar, idxs[k], 0))
            ii.append(iik)
            sc.append(jnp.sum(jnp.where(rowc == iik, s, 0.0)))

        lane128 = lax.broadcasted_iota(jnp.int32, (1, 128), 1)
        m4 = lane128 < 4
        sv = jnp.zeros((1, 128), jnp.float32)
        for k in range(4):
            sv = sv + jnp.where(lane128 == k, sc[k], 0.0)

        mn = jnp.min(jnp.where(m4, sv, BIGF))
        mx = jnp.max(jnp.where(m4, sv, -BIGF))
        neq = mn != mx
        denom = jnp.where(neq, mx - mn, jnp.float32(1.0))
        nor = jnp.where(neq, (mx - sv) / denom, sv)
        nor = jnp.where(m4, nor, 0.0)
        tot = jnp.sum(nor)
        p = nor / tot
        pn0 = p != 0.0
        ent = jnp.sum(jnp.where(m4 & pn0,
                                p * jnp.log(jnp.where(pn0, p, 1.0)), 0.0))
        Es = (-1.0 / _LN4) * ent

        rep = jnp.where(m4, jnp.float32(0.05), 0.0)
        totr = jnp.sum(rep)
        q = rep / totr
        qn0 = q != 0.0
        entr = jnp.sum(jnp.where(m4 & qn0,
                                 q * jnp.log(jnp.where(qn0, q, 1.0)), 0.0))
        Er = (-1.0 / _LN4) * entr

        alpha = (1.0 - Es) / (2.0 - Es - Er)
        beta = (1.0 - Er) / (2.0 - Es - Er)
        xin = jnp.where(m4, alpha * sv + beta * rep, 0.0)
        xi = xin / jnp.sum(xin)

        w = jnp.zeros((1, 128), jnp.float32)
        for k in range(4):
            xik = jnp.sum(jnp.where(lane128 == k, xi, 0.0))
            w = w + jnp.where(lane128 == k, xik, 0.0)
            w = w + jnp.where(lane128 == 4 + k, ii[k].astype(jnp.float32), 0.0)
        w_ref[...] = w


def _stage1(x2d):
    return pl.pallas_call(
        _stage1_body,
        grid=(_NBLK,),
        in_specs=[pl.BlockSpec((_DBLK, _N), lambda i: (i, 0))],
        out_specs=pl.BlockSpec((1, 128), lambda i: (0, 0)),
        out_shape=jax.ShapeDtypeStruct((1, 128), jnp.float32),
        scratch_shapes=[pltpu.VMEM((_N, _N), jnp.float32),
                        pltpu.VMEM((1, _N), jnp.float32)],
    )(x2d)


def _stage2(x1d, wflat, iflat):
    mesh = plsc.VectorSubcoreMesh(core_axis_name="c", subcore_axis_name="s")

    @functools.partial(
        pl.kernel,
        mesh=mesh,
        out_type=jax.ShapeDtypeStruct((_D,), jnp.float32),
        compiler_params=pltpu.CompilerParams(needs_layout_passes=False),
        scratch_types=[pltpu.VMEM((_CH * _N,), jnp.float32),
                       pltpu.VMEM((_CH,), jnp.float32),
                       pltpu.VMEM((4 * 16,), jnp.float32),
                       pltpu.VMEM((4 * 16,), jnp.int32)],
    )
    def k(x_hbm, w_hbm, i_hbm, out_hbm, xv, ov, wv, iv):
        wid = lax.axis_index("s") * 2 + lax.axis_index("c")
        pltpu.sync_copy(w_hbm, wv)
        pltpu.sync_copy(i_hbm, iv)
        for c in range(_RPW // _CH):
            base = wid * _RPW + c * _CH
            pltpu.sync_copy(x_hbm.at[pl.ds(base * _N, _CH * _N)], xv)

            def body(gi, carry):
                flat = (gi * 16 + lax.iota(jnp.int32, 16)) * _N
                acc = None
                for j in range(4):
                    v = plsc.load_gather(xv, [flat + iv[pl.ds(j * 16, 16)]])
                    t = v * wv[pl.ds(j * 16, 16)]
                    acc = t if acc is None else acc + t
                ov[pl.ds(gi * 16, 16)] = acc
                return carry

            lax.fori_loop(0, _CH // 16, body, 0)
            pltpu.sync_copy(ov, out_hbm.at[pl.ds(base, _CH)])

    return k(x1d, wflat, iflat)


def kernel(input):
    x2d = input.reshape(_D, _N)
    w128 = _stage1(x2d)
    w4 = w128[0, 0:4]
    i4 = w128[0, 4:8].astype(jnp.int32)
    wflat = jnp.broadcast_to(w4[:, None], (4, 16)).reshape(4 * 16)
    iflat = jnp.broadcast_to(i4[:, None], (4, 16)).reshape(4 * 16)
    out = _stage2(input.reshape(_D * _N), wflat, iflat)
    return out.reshape(1, _D, 1)
